# baseline (speedup 1.0000x reference)
import jax
import jax.numpy as jnp
from jax.experimental import pallas as pl

_LEVELS = (8, 8, 8, 5, 5, 5)
_BASIS = (1.0, 8.0, 64.0, 512.0, 2560.0, 12800.0)
_D = len(_LEVELS)
_B, _S = 32, 1024
_RNE = 1.5 * 2.0**23


def _fsq_body(z_ref, q_ref, idx_ref, li_ref):
    acc = jnp.zeros((_B, _S), jnp.float32)
    for j in range(_D):
        x = z_ref[j]
        act = (jnp.tanh(x) + 1.0) * 0.5
        y = act * jnp.float32(_LEVELS[j] - 1)
        lif = (y + _RNE) - _RNE
        q_ref[j] = (lif / jnp.float32(_LEVELS[j] - 1)) * 2.0 - 1.0
        li_ref[j] = lif.astype(jnp.int32)
        acc = acc + lif * jnp.float32(_BASIS[j])
    idx_ref[...] = acc.astype(jnp.int32)


_fsq = pl.pallas_call(
    _fsq_body,
    out_shape=[
        jax.ShapeDtypeStruct((_D, _B, _S), jnp.float32),
        jax.ShapeDtypeStruct((_B, _S), jnp.int32),
        jax.ShapeDtypeStruct((_D, _B, _S), jnp.int32),
    ],
)


def kernel(z):
    q, idx, li = _fsq(z.transpose(2, 0, 1))
    return q.transpose(1, 2, 0), idx, li.transpose(1, 2, 0)


# speedup vs baseline: 1.0242x; 1.0242x over previous
import jax
import jax.numpy as jnp
from jax.experimental import pallas as pl
from jax.experimental.pallas import tpu as pltpu

_LEVELS = (8, 8, 8, 5, 5, 5)
_BASIS = (1.0, 8.0, 64.0, 512.0, 2560.0, 12800.0)
_D = len(_LEVELS)
_B, _S = 32, 1024
_RNE = 1.5 * 2.0**23


def _fsq_body(z_ref, q_ref, idx_ref, li_ref):
    acc = jnp.zeros((_B, _S), jnp.float32)
    for j in range(_D):
        x = z_ref[j]
        act = (jnp.tanh(x) + 1.0) * 0.5
        y = act * jnp.float32(_LEVELS[j] - 1)
        lif = (y + _RNE) - _RNE
        q_ref[j] = (lif / jnp.float32(_LEVELS[j] - 1)) * 2.0 - 1.0
        li_ref[j] = lif.astype(jnp.int32)
        acc = acc + lif * jnp.float32(_BASIS[j])
    idx_ref[...] = acc.astype(jnp.int32)


_fsq = pl.pallas_call(
    _fsq_body,
    out_shape=[
        jax.ShapeDtypeStruct((_D, _B, _S), jnp.float32),
        jax.ShapeDtypeStruct((_B, _S), jnp.int32),
        jax.ShapeDtypeStruct((_D, _B, _S), jnp.int32),
    ],
)


def kernel(z):
    zp = pltpu.with_memory_space_constraint(
        z.transpose(2, 0, 1), pltpu.MemorySpace.HBM)
    q, idx, li = _fsq(zp)
    return q.transpose(1, 2, 0), idx, li.transpose(1, 2, 0)
